# Initial kernel scaffold; baseline (speedup 1.0000x reference)
import jax
import jax.numpy as jnp
from jax.experimental import pallas as pl


def kernel(hidden_states, gate_w, e_score_correction_bias, w_gate_up, w_down, shared_gate_up, shared_down):
    raise NotImplementedError("write your pallas kernel here")


# speedup vs baseline: 1.2553x; 1.2553x over previous
import functools
import jax
import jax.numpy as jnp
from jax.experimental import pallas as pl
from jax.experimental.pallas import tpu as pltpu

TOP_K = 2
N_GROUP = 4
TOPK_GROUP = 2
ROUTED_SCALING = 2.5


def _router_body(hidden_ref, gate_ref, bias_ref, combine_ref, *, T, E):
    h = hidden_ref[...]
    logits = jnp.dot(h, gate_ref[...], preferred_element_type=jnp.float32)
    scores = jax.nn.sigmoid(logits)
    s_choice = scores + bias_ref[...]
    sh1 = jnp.concatenate([s_choice[:, 1:], s_choice[:, :1]], axis=1)
    gs8 = s_choice + sh1
    lane_e = jax.lax.broadcasted_iota(jnp.int32, (T, E), 1)
    even = (lane_e % 2) == 0
    gfull = jnp.where(even, gs8, -1e30)
    rank_g = jnp.zeros((T, E), jnp.int32)
    for hg in range(0, E, 2):
        sh = gfull[:, hg:hg + 1]
        rank_g += ((sh > gfull) | ((sh == gfull) & (hg < lane_e))).astype(jnp.int32)
    mask_even = jnp.where((rank_g < TOPK_GROUP) & even, 1.0, 0.0)
    mprev = jnp.concatenate([mask_even[:, -1:], mask_even[:, :-1]], axis=1)
    mask_e = jnp.where(even, mask_even, mprev)
    tmp = jnp.where(mask_e > 0, s_choice, 0.0)
    lane_e = jax.lax.broadcasted_iota(jnp.int32, (T, E), 1)
    rank_e = jnp.zeros((T, E), jnp.int32)
    for he in range(E):
        sh = tmp[:, he:he + 1]
        rank_e += ((sh > tmp) | ((sh == tmp) & (he < lane_e))).astype(jnp.int32)
    sel = rank_e < TOP_K
    w = jnp.where(sel, scores, 0.0)
    denom = jnp.sum(w, axis=1, keepdims=True) + 1e-20
    combine_ref[...] = (w / denom) * ROUTED_SCALING


def _routed_body(hidden_ref, combine_ref, wgu_ref, wd_ref, out_ref, *, E, I, TB):
    e = pl.program_id(0)
    t = pl.program_id(1)
    hc = hidden_ref[...]
    gu = jnp.dot(hc, wgu_ref[0], preferred_element_type=jnp.float32)
    g = gu[:, :I]
    u = gu[:, I:]
    act = (g * jax.nn.sigmoid(g) * u).astype(jnp.bfloat16)
    contrib = jnp.dot(act, wd_ref[0], preferred_element_type=jnp.float32)
    lane_e = jax.lax.broadcasted_iota(jnp.int32, (TB, E), 1)
    cw = jnp.sum(combine_ref[...] * (lane_e == e).astype(jnp.float32),
                 axis=1, keepdims=True)
    sl = pl.ds(t * TB, TB)

    @pl.when(e == 0)
    def _init():
        out_ref[sl, :] = contrib * cw

    @pl.when(e > 0)
    def _acc():
        out_ref[sl, :] = out_ref[sl, :] + contrib * cw


def _shared_body(hidden_ref, routed_ref, sgu_ref, sd_ref, out_ref, *, I_sh):
    h = hidden_ref[...]
    gu = jnp.dot(h, sgu_ref[...], preferred_element_type=jnp.float32)
    g = gu[:, :I_sh]
    u = gu[:, I_sh:]
    act = (g * jax.nn.sigmoid(g) * u).astype(jnp.bfloat16)
    sh = jnp.dot(act, sd_ref[...], preferred_element_type=jnp.float32)
    out_ref[...] = routed_ref[...] + sh


def kernel(hidden_states, gate_w, e_score_correction_bias, w_gate_up, w_down,
           shared_gate_up, shared_down):
    T, H = hidden_states.shape
    E = gate_w.shape[1]
    I = w_down.shape[1]
    I_sh = shared_down.shape[0]

    bias = e_score_correction_bias.reshape(1, E)
    h_bf = hidden_states.astype(jnp.bfloat16)
    wgu_bf = w_gate_up.astype(jnp.bfloat16)
    wd_bf = w_down.astype(jnp.bfloat16)
    sgu_bf = shared_gate_up.astype(jnp.bfloat16)
    sd_bf = shared_down.astype(jnp.bfloat16)

    combine = pl.pallas_call(
        functools.partial(_router_body, T=T, E=E),
        in_specs=[
            pl.BlockSpec((T, H), lambda: (0, 0)),
            pl.BlockSpec((H, E), lambda: (0, 0)),
            pl.BlockSpec((1, E), lambda: (0, 0)),
        ],
        out_specs=pl.BlockSpec((T, E), lambda: (0, 0)),
        out_shape=jax.ShapeDtypeStruct((T, E), jnp.float32),
    )(hidden_states, gate_w, bias)

    TB = 512
    NT = T // TB
    routed = pl.pallas_call(
        functools.partial(_routed_body, E=E, I=I, TB=TB),
        grid=(E, NT),
        in_specs=[
            pl.BlockSpec((TB, H), lambda e, t: (t, 0)),
            pl.BlockSpec((TB, E), lambda e, t: (t, 0)),
            pl.BlockSpec((1, H, 2 * I), lambda e, t: (e, 0, 0)),
            pl.BlockSpec((1, I, H), lambda e, t: (e, 0, 0)),
        ],
        out_specs=pl.BlockSpec((T, H), lambda e, t: (0, 0)),
        out_shape=jax.ShapeDtypeStruct((T, H), jnp.float32),
    )(h_bf, combine, wgu_bf, wd_bf)

    TB2 = 512
    NT2 = T // TB2
    out = pl.pallas_call(
        functools.partial(_shared_body, I_sh=I_sh),
        grid=(NT2,),
        in_specs=[
            pl.BlockSpec((TB2, H), lambda t: (t, 0)),
            pl.BlockSpec((TB2, H), lambda t: (t, 0)),
            pl.BlockSpec((H, 2 * I_sh), lambda t: (0, 0)),
            pl.BlockSpec((I_sh, H), lambda t: (0, 0)),
        ],
        out_specs=pl.BlockSpec((TB2, H), lambda t: (t, 0)),
        out_shape=jax.ShapeDtypeStruct((T, H), jnp.float32),
    )(h_bf, routed, sgu_bf, sd_bf)
    return out
